# initial kernel scaffold (unmeasured)
import jax
import jax.numpy as jnp
from jax import lax
from jax.experimental import pallas as pl
from jax.experimental.pallas import tpu as pltpu


def kernel(
    x,
):
    def body(*refs):
        pass

    out_shape = jax.ShapeDtypeStruct(..., jnp.float32)
    return pl.pallas_call(body, out_shape=out_shape)(...)


# baseline (device time: 48349 ns/iter reference)
import functools

import jax
import jax.numpy as jnp
from jax import lax
from jax.experimental import pallas as pl
from jax.experimental.pallas import tpu as pltpu

N_Y = 4


def kernel(x):
    _, m, n_tot = x.shape
    chunk = n_tot // N_Y

    def body(x_ref, out_ref, comm_ref, send_sems, recv_sems):
        my_x = lax.axis_index("x")
        my_y = lax.axis_index("y")
        my_z = lax.axis_index("z")
        left = (my_x, (my_y + N_Y - 1) % N_Y, my_z)
        right = (my_x, (my_y + 1) % N_Y, my_z)

        barrier_sem = pltpu.get_barrier_semaphore()
        for nbr in (left, right):
            pl.semaphore_signal(
                barrier_sem, inc=1,
                device_id=nbr, device_id_type=pl.DeviceIdType.MESH,
            )
        pl.semaphore_wait(barrier_sem, 2)

        c0 = (my_y + N_Y - 1) % N_Y
        comm_ref[0, :, :] = x_ref[0, :, pl.ds(c0 * chunk, chunk)]

        for s in range(N_Y - 1):
            rdma = pltpu.make_async_remote_copy(
                src_ref=comm_ref.at[s],
                dst_ref=comm_ref.at[s + 1],
                send_sem=send_sems.at[s],
                recv_sem=recv_sems.at[s],
                device_id=right,
                device_id_type=pl.DeviceIdType.MESH,
            )
            rdma.start()
            rdma.wait()
            c = (my_y + 2 * N_Y - s - 2) % N_Y
            add = x_ref[0, :, pl.ds(c * chunk, chunk)]
            if s < N_Y - 2:
                comm_ref[s + 1, :, :] = comm_ref[s + 1, :, :] + add
            else:
                out_ref[:, :] = comm_ref[s + 1, :, :] + add

        @functools.partial(
            pl.run_scoped, second_barrier=pltpu.SemaphoreType.REGULAR
        )
        def _(second_barrier):
            for nbr in (left, right):
                pl.semaphore_signal(
                    second_barrier, inc=1,
                    device_id=nbr, device_id_type=pl.DeviceIdType.MESH,
                )
            pl.semaphore_wait(second_barrier, 2)

    return pl.pallas_call(
        body,
        out_shape=jax.ShapeDtypeStruct((m, chunk), jnp.float32),
        in_specs=[pl.BlockSpec(memory_space=pltpu.VMEM)],
        out_specs=pl.BlockSpec(memory_space=pltpu.VMEM),
        scratch_shapes=[
            pltpu.VMEM((N_Y, m, chunk), jnp.float32),
            pltpu.SemaphoreType.DMA((N_Y - 1,)),
            pltpu.SemaphoreType.DMA((N_Y - 1,)),
        ],
        compiler_params=pltpu.CompilerParams(collective_id=0),
    )(x)


# device time: 39458 ns/iter; 1.2253x vs baseline; 1.2253x over previous
import functools

import jax
import jax.numpy as jnp
from jax import lax
from jax.experimental import pallas as pl
from jax.experimental.pallas import tpu as pltpu

N_Y = 4


def kernel(x):
    _, m, n_tot = x.shape
    chunk = n_tot // N_Y
    half = chunk // 2

    def body(x_ref, out_ref, comm_ref, send_sems, recv_sems, ex_sems):
        my_x = lax.axis_index("x")
        my_y = lax.axis_index("y")
        my_z = lax.axis_index("z")
        left = (my_x, (my_y + N_Y - 1) % N_Y, my_z)
        right = (my_x, (my_y + 1) % N_Y, my_z)
        partner = (1 - my_x, my_y, my_z)

        barrier_sem = pltpu.get_barrier_semaphore()
        for nbr in (left, right, partner):
            pl.semaphore_signal(
                barrier_sem, inc=1,
                device_id=nbr, device_id_type=pl.DeviceIdType.MESH,
            )
        pl.semaphore_wait(barrier_sem, 3)

        def col(c):
            return c * chunk + my_x * half

        c0 = (my_y + N_Y - 1) % N_Y
        comm_ref[0, :, :] = x_ref[0, :, pl.ds(col(c0), half)]

        for s in range(N_Y - 1):
            rdma = pltpu.make_async_remote_copy(
                src_ref=comm_ref.at[s],
                dst_ref=comm_ref.at[s + 1],
                send_sem=send_sems.at[s],
                recv_sem=recv_sems.at[s],
                device_id=right,
                device_id_type=pl.DeviceIdType.MESH,
            )
            rdma.start()
            rdma.wait()
            c = (my_y + 2 * N_Y - s - 2) % N_Y
            add = x_ref[0, :, pl.ds(col(c), half)]
            if s < N_Y - 2:
                comm_ref[s + 1, :, :] = comm_ref[s + 1, :, :] + add
            else:
                out_ref[:, pl.ds(my_x * half, half)] = (
                    comm_ref[s + 1, :, :] + add
                )

        ex = pltpu.make_async_remote_copy(
            src_ref=out_ref.at[:, pl.ds(my_x * half, half)],
            dst_ref=out_ref.at[:, pl.ds(my_x * half, half)],
            send_sem=ex_sems.at[0],
            recv_sem=ex_sems.at[1],
            device_id=partner,
            device_id_type=pl.DeviceIdType.MESH,
        )
        ex.start()
        ex.wait()

        @functools.partial(
            pl.run_scoped, second_barrier=pltpu.SemaphoreType.REGULAR
        )
        def _(second_barrier):
            for nbr in (left, right, partner):
                pl.semaphore_signal(
                    second_barrier, inc=1,
                    device_id=nbr, device_id_type=pl.DeviceIdType.MESH,
                )
            pl.semaphore_wait(second_barrier, 3)

    return pl.pallas_call(
        body,
        out_shape=jax.ShapeDtypeStruct((m, chunk), jnp.float32),
        in_specs=[pl.BlockSpec(memory_space=pltpu.VMEM)],
        out_specs=pl.BlockSpec(memory_space=pltpu.VMEM),
        scratch_shapes=[
            pltpu.VMEM((N_Y, m, half), jnp.float32),
            pltpu.SemaphoreType.DMA((N_Y - 1,)),
            pltpu.SemaphoreType.DMA((N_Y - 1,)),
            pltpu.SemaphoreType.DMA((2,)),
        ],
        compiler_params=pltpu.CompilerParams(collective_id=0),
    )(x)


# device time: 32433 ns/iter; 1.4907x vs baseline; 1.2166x over previous
import functools

import jax
import jax.numpy as jnp
from jax import lax
from jax.experimental import pallas as pl
from jax.experimental.pallas import tpu as pltpu

N_Y = 4


def kernel(x):
    _, m, n_tot = x.shape
    chunk = n_tot // N_Y
    half = chunk // 2
    sub = half // 2

    def body(
        x_ref, out_ref,
        comm_a, comm_b,
        sends_a, recvs_a, sends_b, recvs_b, ex_sems,
    ):
        my_x = lax.axis_index("x")
        my_y = lax.axis_index("y")
        my_z = lax.axis_index("z")
        left = (my_x, (my_y + N_Y - 1) % N_Y, my_z)
        right = (my_x, (my_y + 1) % N_Y, my_z)
        partner = (1 - my_x, my_y, my_z)

        barrier_sem = pltpu.get_barrier_semaphore()
        for nbr in (left, right, partner):
            pl.semaphore_signal(
                barrier_sem, inc=1,
                device_id=nbr, device_id_type=pl.DeviceIdType.MESH,
            )
        pl.semaphore_wait(barrier_sem, 3)

        def subcol(c, u):
            return c * chunk + my_x * half + u * sub

        def ring_rdma(comm, sends, recvs, s):
            return pltpu.make_async_remote_copy(
                src_ref=comm.at[s],
                dst_ref=comm.at[s + 1],
                send_sem=sends.at[s],
                recv_sem=recvs.at[s],
                device_id=right,
                device_id_type=pl.DeviceIdType.MESH,
            )

        rd = [
            [ring_rdma(comm_a, sends_a, recvs_a, s) for s in range(N_Y - 1)],
            [ring_rdma(comm_b, sends_b, recvs_b, s) for s in range(N_Y - 1)],
        ]
        ex = [
            pltpu.make_async_remote_copy(
                src_ref=out_ref.at[:, pl.ds(my_x * half + u * sub, sub)],
                dst_ref=out_ref.at[:, pl.ds(my_x * half + u * sub, sub)],
                send_sem=ex_sems.at[2 * u],
                recv_sem=ex_sems.at[2 * u + 1],
                device_id=partner,
                device_id_type=pl.DeviceIdType.MESH,
            )
            for u in range(2)
        ]
        comms = (comm_a, comm_b)

        c0 = (my_y + N_Y - 1) % N_Y
        for u in range(2):
            comms[u][0, :, :] = x_ref[0, :, pl.ds(subcol(c0, u), sub)]
            rd[u][0].start()

        for s in range(N_Y - 1):
            c = (my_y + 2 * N_Y - s - 2) % N_Y
            for u in range(2):
                rd[u][s].wait_recv()
                add = x_ref[0, :, pl.ds(subcol(c, u), sub)]
                if s < N_Y - 2:
                    comms[u][s + 1, :, :] = comms[u][s + 1, :, :] + add
                    rd[u][s + 1].start()
                else:
                    out_ref[:, pl.ds(my_x * half + u * sub, sub)] = (
                        comms[u][s + 1, :, :] + add
                    )
                    ex[u].start()

        ex[0].wait_recv()
        ex[1].wait_recv()
        for u in range(2):
            for s in range(N_Y - 1):
                rd[u][s].wait_send()
            ex[u].wait_send()

        @functools.partial(
            pl.run_scoped, second_barrier=pltpu.SemaphoreType.REGULAR
        )
        def _(second_barrier):
            for nbr in (left, right, partner):
                pl.semaphore_signal(
                    second_barrier, inc=1,
                    device_id=nbr, device_id_type=pl.DeviceIdType.MESH,
                )
            pl.semaphore_wait(second_barrier, 3)

    return pl.pallas_call(
        body,
        out_shape=jax.ShapeDtypeStruct((m, chunk), jnp.float32),
        in_specs=[pl.BlockSpec(memory_space=pltpu.VMEM)],
        out_specs=pl.BlockSpec(memory_space=pltpu.VMEM),
        scratch_shapes=[
            pltpu.VMEM((N_Y, m, sub), jnp.float32),
            pltpu.VMEM((N_Y, m, sub), jnp.float32),
            pltpu.SemaphoreType.DMA((N_Y - 1,)),
            pltpu.SemaphoreType.DMA((N_Y - 1,)),
            pltpu.SemaphoreType.DMA((N_Y - 1,)),
            pltpu.SemaphoreType.DMA((N_Y - 1,)),
            pltpu.SemaphoreType.DMA((4,)),
        ],
        compiler_params=pltpu.CompilerParams(collective_id=0),
    )(x)


# device time: 31053 ns/iter; 1.5570x vs baseline; 1.0444x over previous
import functools

import jax
import jax.numpy as jnp
from jax import lax
from jax.experimental import pallas as pl
from jax.experimental.pallas import tpu as pltpu

N_Y = 4
N_SUB = 4


def kernel(x):
    _, m, n_tot = x.shape
    chunk = n_tot // N_Y
    half = chunk // 2
    subm = m // N_SUB

    def body(x_ref, out_ref, comm, sends, recvs, ex_sems):
        my_x = lax.axis_index("x")
        my_y = lax.axis_index("y")
        my_z = lax.axis_index("z")
        left = (my_x, (my_y + N_Y - 1) % N_Y, my_z)
        right = (my_x, (my_y + 1) % N_Y, my_z)
        partner = (1 - my_x, my_y, my_z)

        barrier_sem = pltpu.get_barrier_semaphore()
        for nbr in (left, right, partner):
            pl.semaphore_signal(
                barrier_sem, inc=1,
                device_id=nbr, device_id_type=pl.DeviceIdType.MESH,
            )
        pl.semaphore_wait(barrier_sem, 3)

        col0 = my_x * half

        def x_slice(c, u):
            return x_ref.at[0, pl.ds(u * subm, subm),
                            pl.ds(c * chunk + col0, half)]

        def ring_rdma(u, s, src):
            return pltpu.make_async_remote_copy(
                src_ref=src,
                dst_ref=comm.at[u, s + 1],
                send_sem=sends.at[u, s],
                recv_sem=recvs.at[u, s],
                device_id=right,
                device_id_type=pl.DeviceIdType.MESH,
            )

        c0 = (my_y + N_Y - 1) % N_Y
        rd = [
            [ring_rdma(u, 0, x_slice(c0, u))]
            + [ring_rdma(u, s, comm.at[u, s]) for s in range(1, N_Y - 1)]
            for u in range(N_SUB)
        ]
        ex = [
            pltpu.make_async_remote_copy(
                src_ref=out_ref.at[pl.ds(u * subm, subm), pl.ds(col0, half)],
                dst_ref=out_ref.at[pl.ds(u * subm, subm), pl.ds(col0, half)],
                send_sem=ex_sems.at[u, 0],
                recv_sem=ex_sems.at[u, 1],
                device_id=partner,
                device_id_type=pl.DeviceIdType.MESH,
            )
            for u in range(N_SUB)
        ]

        for u in range(N_SUB):
            rd[u][0].start()

        for s in range(N_Y - 1):
            c = (my_y + 2 * N_Y - s - 2) % N_Y
            for u in range(N_SUB):
                rd[u][s].wait_recv()
                add = x_slice(c, u)[...]
                if s < N_Y - 2:
                    comm[u, s + 1, :, :] = comm[u, s + 1, :, :] + add
                    rd[u][s + 1].start()
                else:
                    out_ref[pl.ds(u * subm, subm), pl.ds(col0, half)] = (
                        comm[u, s + 1, :, :] + add
                    )
                    ex[u].start()

        for u in range(N_SUB):
            ex[u].wait_recv()
        for u in range(N_SUB):
            for s in range(N_Y - 1):
                rd[u][s].wait_send()
            ex[u].wait_send()

        @functools.partial(
            pl.run_scoped, second_barrier=pltpu.SemaphoreType.REGULAR
        )
        def _(second_barrier):
            for nbr in (left, right, partner):
                pl.semaphore_signal(
                    second_barrier, inc=1,
                    device_id=nbr, device_id_type=pl.DeviceIdType.MESH,
                )
            pl.semaphore_wait(second_barrier, 3)

    return pl.pallas_call(
        body,
        out_shape=jax.ShapeDtypeStruct((m, chunk), jnp.float32),
        in_specs=[pl.BlockSpec(memory_space=pltpu.VMEM)],
        out_specs=pl.BlockSpec(memory_space=pltpu.VMEM),
        scratch_shapes=[
            pltpu.VMEM((N_SUB, N_Y, subm, half), jnp.float32),
            pltpu.SemaphoreType.DMA((N_SUB, N_Y - 1)),
            pltpu.SemaphoreType.DMA((N_SUB, N_Y - 1)),
            pltpu.SemaphoreType.DMA((N_SUB, 2)),
        ],
        compiler_params=pltpu.CompilerParams(collective_id=0),
    )(x)


# device time: 24718 ns/iter; 1.9560x vs baseline; 1.2563x over previous
import jax
import jax.numpy as jnp
from jax import lax
from jax.experimental import pallas as pl
from jax.experimental.pallas import tpu as pltpu

N_Y = 4
N_SUB = 4
N_EX = 3


def kernel(x):
    _, m, n_tot = x.shape
    chunk = n_tot // N_Y
    quart = chunk // 4
    subm = m // N_SUB

    def body(x_ref, out_ref, comm, sends, recvs, ex_sends, ex_recvs):
        my_x = lax.axis_index("x")
        my_y = lax.axis_index("y")
        my_z = lax.axis_index("z")
        zp = my_z - (my_z % 2) * 2 + 1
        left = (my_x, (my_y + N_Y - 1) % N_Y, my_z)
        right = (my_x, (my_y + 1) % N_Y, my_z)
        partners = (
            (my_x, my_y, zp),
            (1 - my_x, my_y, my_z),
            (1 - my_x, my_y, zp),
        )

        barrier_sem = pltpu.get_barrier_semaphore()
        for nbr in (left, right) + partners:
            pl.semaphore_signal(
                barrier_sem, inc=1,
                device_id=nbr, device_id_type=pl.DeviceIdType.MESH,
            )
        pl.semaphore_wait(barrier_sem, 5)

        q_own = 2 * my_x + (my_z % 2)
        col0 = q_own * quart

        def x_slice(c, u):
            return x_ref.at[0, pl.ds(u * subm, subm),
                            pl.ds(c * chunk + col0, quart)]

        def ring_rdma(u, s, src):
            return pltpu.make_async_remote_copy(
                src_ref=src,
                dst_ref=comm.at[u, s + 1],
                send_sem=sends.at[u, s],
                recv_sem=recvs.at[u, s],
                device_id=right,
                device_id_type=pl.DeviceIdType.MESH,
            )

        c0 = (my_y + N_Y - 1) % N_Y
        rd = [
            [ring_rdma(u, 0, x_slice(c0, u))]
            + [ring_rdma(u, s, comm.at[u, s]) for s in range(1, N_Y - 1)]
            for u in range(N_SUB)
        ]
        ex = [
            [
                pltpu.make_async_remote_copy(
                    src_ref=out_ref.at[pl.ds(u * subm, subm),
                                       pl.ds(col0, quart)],
                    dst_ref=out_ref.at[pl.ds(u * subm, subm),
                                       pl.ds(col0, quart)],
                    send_sem=ex_sends.at[k, u],
                    recv_sem=ex_recvs.at[k, u],
                    device_id=partners[k],
                    device_id_type=pl.DeviceIdType.MESH,
                )
                for u in range(N_SUB)
            ]
            for k in range(N_EX)
        ]

        for u in range(N_SUB):
            rd[u][0].start()

        for s in range(N_Y - 1):
            c = (my_y + 2 * N_Y - s - 2) % N_Y
            for u in range(N_SUB):
                rd[u][s].wait_recv()
                add = x_slice(c, u)[...]
                if s < N_Y - 2:
                    comm[u, s + 1, :, :] = comm[u, s + 1, :, :] + add
                    rd[u][s + 1].start()
                else:
                    out_ref[pl.ds(u * subm, subm), pl.ds(col0, quart)] = (
                        comm[u, s + 1, :, :] + add
                    )
                    for k in range(N_EX):
                        ex[k][u].start()

        for k in range(N_EX):
            for u in range(N_SUB):
                ex[k][u].wait_recv()
        for u in range(N_SUB):
            for s in range(N_Y - 1):
                rd[u][s].wait_send()
        for k in range(N_EX):
            for u in range(N_SUB):
                ex[k][u].wait_send()

    return pl.pallas_call(
        body,
        out_shape=jax.ShapeDtypeStruct((m, chunk), jnp.float32),
        in_specs=[pl.BlockSpec(memory_space=pltpu.VMEM)],
        out_specs=pl.BlockSpec(memory_space=pltpu.VMEM),
        scratch_shapes=[
            pltpu.VMEM((N_SUB, N_Y, subm, quart), jnp.float32),
            pltpu.SemaphoreType.DMA((N_SUB, N_Y - 1)),
            pltpu.SemaphoreType.DMA((N_SUB, N_Y - 1)),
            pltpu.SemaphoreType.DMA((N_EX, N_SUB)),
            pltpu.SemaphoreType.DMA((N_EX, N_SUB)),
        ],
        compiler_params=pltpu.CompilerParams(collective_id=0),
    )(x)
